# baseline (speedup 1.0000x reference)
import jax
import jax.numpy as jnp
from jax.experimental import pallas as pl

LENS_A = [1500, 900, 2100, 1100]
LENS_B = [500, 1100, 300, 900]
D = 256
N_ORIG = sum(LENS_A) + sum(LENS_B)
N = 8448

_ORDERED_LENS = [LENS_A[0], LENS_B[0], LENS_A[1], LENS_B[1],
                 LENS_A[2], LENS_B[2], LENS_A[3], LENS_B[3]]
_OFFSETS = []
_off = 0
for _l in _ORDERED_LENS:
    _OFFSETS.append(_off)
    _off += _l

_SEG_STARTS = [0, 2000, 4000, 6400]

MASK_TILE_R = 768


def _pack_kernel(a0, a1, a2, a3, b0, b1, b2, b3, out_ref):
    ins = [a0, b0, a1, b1, a2, b2, a3, b3]
    for ref, off, l in zip(ins, _OFFSETS, _ORDERED_LENS):
        out_ref[0, off:off + l, :] = ref[0]
    out_ref[0, N_ORIG:N, :] = jnp.zeros((N - N_ORIG, D), jnp.float32)


def _mask_kernel(out_ref):
    i = pl.program_id(0)
    q = jax.lax.broadcasted_iota(jnp.int32, (MASK_TILE_R, 1), 0) + i * MASK_TILE_R
    k = jax.lax.broadcasted_iota(jnp.int32, (1, N), 1)

    def seg_id(p):
        s = jnp.zeros(p.shape, jnp.int32)
        for b in _SEG_STARTS[1:]:
            s = s + (p >= b).astype(jnp.int32)
        return s

    m = (seg_id(q) == seg_id(k)) & (q < N_ORIG) & (k < N_ORIG)
    out_ref[...] = m


def kernel(a0, a1, a2, a3, b0, b1, b2, b3):
    packed = pl.pallas_call(
        _pack_kernel,
        out_shape=jax.ShapeDtypeStruct((1, N, D), jnp.float32),
    )(a0, a1, a2, a3, b0, b1, b2, b3)

    mask = pl.pallas_call(
        _mask_kernel,
        grid=(N // MASK_TILE_R,),
        out_specs=pl.BlockSpec((MASK_TILE_R, N), lambda i: (i, 0)),
        out_shape=jax.ShapeDtypeStruct((N, N), jnp.bool_),
    )()
    return packed, mask
